# baseline (speedup 1.0000x reference)
import functools

import jax
import jax.numpy as jnp
from jax import lax
from jax.experimental import pallas as pl
from jax.experimental.pallas import tpu as pltpu
from jax.experimental.pallas import tpu_sc as plsc

_H = 1000
_ST = _H // 8


def _sc_body(xt_hbm, out_hbm, idx_v, slab_v, s):
    b = idx_v.shape[0]
    nchunk = b // 16
    info = plsc.get_sparse_core_info()
    wid = lax.axis_index("c") * info.num_subcores + lax.axis_index("s")
    nw = info.num_cores * info.num_subcores
    st_lo = (_ST * wid) // nw
    st_hi = (_ST * (wid + 1)) // nw

    one = jnp.full((16,), 1.0, jnp.float32)
    zero = jnp.zeros((16,), jnp.float32)

    def _j_loop(j, _):
        pltpu.sync_copy(xt_hbm.at[j], idx_v)

        def _st_loop(st, __):
            c0 = st * 8

            def _chunk(k, ___):
                v = idx_v[pl.ds(k * 16, 16)]
                u = v - c0
                for r in range(8):
                    slab_v[r, pl.ds(k * 16, 16)] = jnp.where(u == r, one, zero)
                return 0
            lax.fori_loop(0, nchunk, _chunk, 0)
            pltpu.sync_copy(slab_v, out_hbm.at[j, pl.ds(st * 8, 8)])
            return 0
        lax.fori_loop(st_lo, st_hi, _st_loop, 0)
        return 0
    lax.fori_loop(0, s, _j_loop, 0)


def kernel(x):
    b, s = x.shape
    xt = x.T.astype(jnp.int32)
    mesh = plsc.VectorSubcoreMesh(core_axis_name="c", subcore_axis_name="s")
    k = pl.kernel(
        functools.partial(_sc_body, s=s),
        mesh=mesh,
        out_type=jax.ShapeDtypeStruct((s, _H, b), jnp.float32),
        scratch_types=[
            pltpu.VMEM((b,), jnp.int32),
            pltpu.VMEM((8, b), jnp.float32),
        ],
    )
    out = k(xt)
    return jnp.transpose(out, (2, 0, 1))


# speedup vs baseline: 3.3213x; 3.3213x over previous
import jax
import jax.numpy as jnp
from jax.experimental import pallas as pl

_H = 1000
_CC = 1000
_LB = 1024


def _body(x_ref, o_ref):
    i = pl.program_id(1)
    idx = x_ref[0, 0, pl.ds(i * _LB, _LB)]
    iota = jax.lax.broadcasted_iota(jnp.int32, (_CC, _LB), 0)
    o_ref[0] = (idx[None, :] == iota).astype(jnp.float32)


def kernel(x):
    b, s = x.shape
    xt = x.T.reshape(s, 1, b).astype(jnp.int32)
    out = pl.pallas_call(
        _body,
        grid=(s, b // _LB),
        in_specs=[pl.BlockSpec((1, 1, b), lambda j, i: (j, 0, 0))],
        out_specs=pl.BlockSpec((1, _CC, _LB), lambda j, i: (j, 0, i)),
        out_shape=jax.ShapeDtypeStruct((s, _H, b), jnp.float32),
    )(xt)
    return jnp.transpose(out, (2, 0, 1))
